# Initial kernel scaffold; baseline (speedup 1.0000x reference)
import jax
import jax.numpy as jnp
from jax.experimental import pallas as pl


def kernel(query, keys, values, top_k, chunk_size, param_feats, Wq, Wk, Wv, w1, b1, w2, b2, gamma, beta):
    raise NotImplementedError("write your pallas kernel here")


# speedup vs baseline: 17.1917x; 17.1917x over previous
import functools

import jax
import jax.numpy as jnp
from jax import lax
from jax.experimental import pallas as pl
from jax.experimental.pallas import tpu as pltpu
from jax.experimental.pallas import tpu_sc as plsc

B, N, D, D1, HID = 64, 32768, 128, 32, 64
K_TOP = 16
ROWS_BLK = 4096
NC, NS, L = 2, 16, 16
NW = NC * NS
QPW = B // NW
_HI = lax.Precision.HIGHEST
_DN_NT = (((1,), (1,)), ((), ()))
_DN_NN = (((1,), (0,)), ((), ()))


def _scores_body(q_ref, wq_ref, wk_ref, keys_ref, out_ref):
    q1 = lax.dot_general(q_ref[...], wq_ref[...], _DN_NT,
                         preferred_element_type=jnp.float32)
    kc = lax.dot_general(keys_ref[...], wk_ref[...], _DN_NT,
                         preferred_element_type=jnp.float32)
    out_ref[...] = lax.dot_general(q1, kc, _DN_NT,
                                   preferred_element_type=jnp.float32)


def _scores(query, Wq, Wk, keys):
    return pl.pallas_call(
        _scores_body,
        grid=(N // ROWS_BLK,),
        in_specs=[
            pl.BlockSpec((B, D), lambda i: (0, 0)),
            pl.BlockSpec((D, D), lambda i: (0, 0)),
            pl.BlockSpec((D, D), lambda i: (0, 0)),
            pl.BlockSpec((ROWS_BLK, D), lambda i: (i, 0)),
        ],
        out_specs=pl.BlockSpec((B, ROWS_BLK), lambda i: (0, i)),
        out_shape=jax.ShapeDtypeStruct((B, N), jnp.float32),
    )(query, Wq, Wk, keys)


def _sc_body(scores_hbm, values_hbm, idx_out, val_out, rows_out,
             srow, idx_v, val_v, rows_v, sem):
    wid = lax.axis_index("s") * NC + lax.axis_index("c")
    for j in range(QPW):
        q = wid * QPW + j
        pltpu.sync_copy(scores_hbm.at[q], srow)

        def lane_max(i, acc):
            return jnp.maximum(acc, srow[pl.ds(i * L, L)])

        lmax = lax.fori_loop(0, N // L, lane_max,
                             jnp.full((L,), -jnp.inf, jnp.float32))
        c = plsc.cummax(-lmax)
        tau_vec = -plsc.cummax(lax.rev(c, (0,)))

        def scan_step(i, carry):
            cur_v, cur_i = carry
            v = srow[pl.ds(i * L, L)]
            hit = jnp.any(v >= tau_vec)

            def merge(c):
                cv, ci = c
                idx = lax.iota(jnp.int32, L) + i * L
                sv, si = plsc.sort_key_val(v, idx, descending=False)
                svr = lax.rev(sv, (0,))
                sir = lax.rev(si, (0,))
                nv = jnp.maximum(svr, cv)
                ni = jnp.where(svr >= cv, sir, ci)
                nv2, ni2 = plsc.sort_key_val(nv, ni, descending=False)
                return (nv2, ni2)

            return lax.cond(hit, merge, lambda c: c, carry)

        cur_v, cur_i = lax.fori_loop(
            0, N // L, scan_step,
            (jnp.full((L,), -jnp.inf, jnp.float32), jnp.zeros((L,), jnp.int32)))

        idx_v[...] = cur_i
        val_v[...] = cur_v
        pltpu.sync_copy(idx_v, idx_out.at[q])
        pltpu.sync_copy(val_v, val_out.at[q])
        pltpu.async_copy(values_hbm.at[idx_v], rows_v, sem).wait()
        pltpu.sync_copy(rows_v, rows_out.at[q])


def _sc_topk_gather(scores, values):
    mesh = plsc.VectorSubcoreMesh(core_axis_name="c", subcore_axis_name="s",
                                  num_cores=NC, num_subcores=NS)
    fn = pl.kernel(
        _sc_body,
        out_type=(jax.ShapeDtypeStruct((B, K_TOP), jnp.int32),
                  jax.ShapeDtypeStruct((B, K_TOP), jnp.float32),
                  jax.ShapeDtypeStruct((B, K_TOP, D), jnp.float32)),
        mesh=mesh,
        compiler_params=pltpu.CompilerParams(needs_layout_passes=False),
        scratch_types=[
            pltpu.VMEM((N,), jnp.float32),
            pltpu.VMEM((K_TOP,), jnp.int32),
            pltpu.VMEM((K_TOP,), jnp.float32),
            pltpu.VMEM((K_TOP, D), jnp.float32),
            pltpu.SemaphoreType.DMA,
        ],
    )
    return fn(scores, values)


def _final_body(rows_ref, sc_ref, pf_ref, wv_ref, w1_ref, b1_ref, w2_ref,
                b2_ref, gamma_ref, beta_ref, out_ref):
    vt = rows_ref[...].reshape(B * K_TOP, D)
    vtop = lax.dot_general(vt, wv_ref[...], _DN_NT,
                           preferred_element_type=jnp.float32, precision=_HI)
    w1 = w1_ref[...]
    h1 = lax.dot_general(vtop, w1[:, :D], _DN_NT,
                         preferred_element_type=jnp.float32, precision=_HI)
    pfh = lax.dot_general(pf_ref[...], w1[:, D:], _DN_NT,
                          preferred_element_type=jnp.float32, precision=_HI)
    pfh = jnp.broadcast_to(pfh[:, None, :], (B, K_TOP, HID)).reshape(
        B * K_TOP, HID)
    h = jnp.maximum(h1 + pfh + b1_ref[...], 0.0)
    h2 = lax.dot_general(h, w2_ref[...], _DN_NT,
                         preferred_element_type=jnp.float32,
                         precision=_HI) + b2_ref[...]
    mu = jnp.mean(h2, axis=1, keepdims=True)
    var = jnp.mean((h2 - mu) * (h2 - mu), axis=1, keepdims=True)
    hn = (h2 - mu) * lax.rsqrt(var + 1e-5) * gamma_ref[...] + beta_ref[...]
    adapted = (vtop + hn).reshape(B, K_TOP, D)

    s = sc_ref[...] * (1.0 / (D ** 0.5))
    e = jnp.exp(s - jnp.max(s, axis=1, keepdims=True))
    w = e / jnp.sum(e, axis=1, keepdims=True)
    out_ref[...] = jnp.sum(adapted * w[:, :, None], axis=1)


def _final(rows, scs, pf, Wv, w1, b1, w2, b2, gamma, beta):
    return pl.pallas_call(
        _final_body,
        out_shape=jax.ShapeDtypeStruct((B, D), jnp.float32),
    )(rows, scs, pf, Wv, w1, b1.reshape(1, HID), w2, b2.reshape(1, D),
      gamma.reshape(1, D), beta.reshape(1, D))


def kernel(query, keys, values, top_k, chunk_size, param_feats,
           Wq, Wk, Wv, w1, b1, w2, b2, gamma, beta):
    if query.ndim == 1:
        query = query[None, :]
    scores = _scores(query, Wq, Wk, keys)
    _, vals, rows = _sc_topk_gather(scores, values)
    return _final(rows, vals, param_feats, Wv, w1, b1, w2, b2, gamma, beta)


# speedup vs baseline: 18.0274x; 1.0486x over previous
import functools

import jax
import jax.numpy as jnp
from jax import lax
from jax.experimental import pallas as pl
from jax.experimental.pallas import tpu as pltpu
from jax.experimental.pallas import tpu_sc as plsc

B, N, D, D1, HID = 64, 32768, 128, 32, 64
K_TOP = 16
ROWS_BLK = 4096
NBLK = N // ROWS_BLK
NC, NS, L = 2, 16, 16
NW = NC * NS
QPW = B // NW
CAP = 2048
_HI = lax.Precision.HIGHEST
_DN_NT = (((1,), (1,)), ((), ()))


def _scores_body(q_ref, wq_ref, wk_ref, keys_ref, out_ref, cmax_ref, acc_ref):
    i = pl.program_id(0)
    q1 = lax.dot_general(q_ref[...], wq_ref[...], _DN_NT,
                         preferred_element_type=jnp.float32)
    kc = lax.dot_general(keys_ref[...], wk_ref[...], _DN_NT,
                         preferred_element_type=jnp.float32)
    s = lax.dot_general(q1, kc, _DN_NT, preferred_element_type=jnp.float32)
    out_ref[...] = s
    m = s[:, :D]
    for k in range(1, ROWS_BLK // D):
        m = jnp.maximum(m, s[:, k * D:(k + 1) * D])

    @pl.when(i == 0)
    def _():
        acc_ref[...] = m

    @pl.when(i > 0)
    def _():
        acc_ref[...] = jnp.maximum(acc_ref[...], m)

    @pl.when(i == NBLK - 1)
    def _():
        cmax_ref[...] = acc_ref[...]


def _scores(query, Wq, Wk, keys):
    return pl.pallas_call(
        _scores_body,
        grid=(NBLK,),
        in_specs=[
            pl.BlockSpec((B, D), lambda i: (0, 0)),
            pl.BlockSpec((D, D), lambda i: (0, 0)),
            pl.BlockSpec((D, D), lambda i: (0, 0)),
            pl.BlockSpec((ROWS_BLK, D), lambda i: (i, 0)),
        ],
        out_specs=(pl.BlockSpec((B, ROWS_BLK), lambda i: (0, i)),
                   pl.BlockSpec((B, D), lambda i: (0, 0))),
        out_shape=(jax.ShapeDtypeStruct((B, N), jnp.float32),
                   jax.ShapeDtypeStruct((B, D), jnp.float32)),
        scratch_shapes=[pltpu.VMEM((B, D), jnp.float32)],
    )(query, Wq, Wk, keys)


def _merge16(cv, ci, v, idx):
    sv, si = plsc.sort_key_val(v, idx, descending=False)
    svr = lax.rev(sv, (0,))
    sir = lax.rev(si, (0,))
    nv = jnp.maximum(svr, cv)
    ni = jnp.where(svr >= cv, sir, ci)
    return plsc.sort_key_val(nv, ni, descending=False)


def _sc_body(scores_hbm, cmax_hbm, values_hbm, idx_out, val_out, rows_out,
             srow0, srow1, cmv, cand_v, cand_i, idx_v, val_v, rows_v,
             sem0, sem1, semg):
    wid = lax.axis_index("s") * NC + lax.axis_index("c")
    q0 = wid * QPW
    srows = (srow0, srow1)
    sems = (sem0, sem1)
    copies = [pltpu.make_async_copy(scores_hbm.at[q0 + j], srows[j], sems[j])
              for j in range(QPW)]
    copies[0].start()
    iota = lax.iota(jnp.int32, L)
    ninf = jnp.full((L,), -jnp.inf, jnp.float32)
    zero_i = jnp.zeros((L,), jnp.int32)

    for j in range(QPW):
        q = q0 + j
        srow = srows[j]
        if j + 1 < QPW:
            copies[j + 1].start()

        pltpu.sync_copy(cmax_hbm.at[q], cmv)
        cv, ci = ninf, zero_i
        for k in range(D // L):
            cv, ci = _merge16(cv, ci, cmv[pl.ds(k * L, L)], zero_i)
        tau_vec = -plsc.cummax(lax.rev(plsc.cummax(-cv), (0,)))

        copies[j].wait()

        def scan_step(i, c_vec):
            v = srow[pl.ds(i * L, L)]
            m = v >= tau_vec
            pre = plsc.cumsum(jnp.where(m, 1, 0))
            didx = jnp.minimum(c_vec + pre - 1, CAP - 1)
            plsc.store_scatter(cand_v, [didx], v, mask=m)
            plsc.store_scatter(cand_i, [didx], iota + i * L, mask=m)
            return c_vec + plsc.all_reduce_population_count(m)

        c_vec = lax.fori_loop(0, N // L, scan_step, zero_i, unroll=8)
        cnt = jnp.max(c_vec)

        def merge_step(k, carry):
            cv, ci = carry
            v = cand_v[pl.ds(k * L, L)]
            ix = cand_i[pl.ds(k * L, L)]
            valid = (iota + k * L) < c_vec
            v = jnp.where(valid, v, -jnp.inf)
            return tuple(_merge16(cv, ci, v, ix))

        cur_v, cur_i = lax.fori_loop(0, (cnt + L - 1) // L, merge_step,
                                     (ninf, zero_i))

        idx_v[...] = cur_i
        val_v[...] = cur_v
        pltpu.sync_copy(idx_v, idx_out.at[q])
        pltpu.sync_copy(val_v, val_out.at[q])
        pltpu.async_copy(values_hbm.at[idx_v], rows_v, semg).wait()
        pltpu.sync_copy(rows_v, rows_out.at[q])


def _sc_topk_gather(scores, cmax, values):
    mesh = plsc.VectorSubcoreMesh(core_axis_name="c", subcore_axis_name="s",
                                  num_cores=NC, num_subcores=NS)
    fn = pl.kernel(
        _sc_body,
        out_type=(jax.ShapeDtypeStruct((B, K_TOP), jnp.int32),
                  jax.ShapeDtypeStruct((B, K_TOP), jnp.float32),
                  jax.ShapeDtypeStruct((B, K_TOP, D), jnp.float32)),
        mesh=mesh,
        compiler_params=pltpu.CompilerParams(needs_layout_passes=False),
        scratch_types=[
            pltpu.VMEM((N,), jnp.float32),
            pltpu.VMEM((N,), jnp.float32),
            pltpu.VMEM((D,), jnp.float32),
            pltpu.VMEM((CAP,), jnp.float32),
            pltpu.VMEM((CAP,), jnp.int32),
            pltpu.VMEM((K_TOP,), jnp.int32),
            pltpu.VMEM((K_TOP,), jnp.float32),
            pltpu.VMEM((K_TOP, D), jnp.float32),
            pltpu.SemaphoreType.DMA,
            pltpu.SemaphoreType.DMA,
            pltpu.SemaphoreType.DMA,
        ],
    )
    return fn(scores, cmax, values)


def _final_body(rows_ref, sc_ref, pf_ref, wv_ref, w1_ref, b1_ref, w2_ref,
                b2_ref, gamma_ref, beta_ref, out_ref):
    vt = rows_ref[...].reshape(B * K_TOP, D)
    vtop = lax.dot_general(vt, wv_ref[...], _DN_NT,
                           preferred_element_type=jnp.float32, precision=_HI)
    w1 = w1_ref[...]
    h1 = lax.dot_general(vtop, w1[:, :D], _DN_NT,
                         preferred_element_type=jnp.float32, precision=_HI)
    pfh = lax.dot_general(pf_ref[...], w1[:, D:], _DN_NT,
                          preferred_element_type=jnp.float32, precision=_HI)
    pfh = jnp.broadcast_to(pfh[:, None, :], (B, K_TOP, HID)).reshape(
        B * K_TOP, HID)
    h = jnp.maximum(h1 + pfh + b1_ref[...], 0.0)
    h2 = lax.dot_general(h, w2_ref[...], _DN_NT,
                         preferred_element_type=jnp.float32,
                         precision=_HI) + b2_ref[...]
    mu = jnp.mean(h2, axis=1, keepdims=True)
    var = jnp.mean((h2 - mu) * (h2 - mu), axis=1, keepdims=True)
    hn = (h2 - mu) * lax.rsqrt(var + 1e-5) * gamma_ref[...] + beta_ref[...]
    adapted = (vtop + hn).reshape(B, K_TOP, D)

    s = sc_ref[...] * (1.0 / (D ** 0.5))
    e = jnp.exp(s - jnp.max(s, axis=1, keepdims=True))
    w = e / jnp.sum(e, axis=1, keepdims=True)
    out_ref[...] = jnp.sum(adapted * w[:, :, None], axis=1)


def _final(rows, scs, pf, Wv, w1, b1, w2, b2, gamma, beta):
    return pl.pallas_call(
        _final_body,
        out_shape=jax.ShapeDtypeStruct((B, D), jnp.float32),
    )(rows, scs, pf, Wv, w1, b1.reshape(1, HID), w2, b2.reshape(1, D),
      gamma.reshape(1, D), beta.reshape(1, D))


def kernel(query, keys, values, top_k, chunk_size, param_feats,
           Wq, Wk, Wv, w1, b1, w2, b2, gamma, beta):
    if query.ndim == 1:
        query = query[None, :]
    scores, cmax = _scores(query, Wq, Wk, keys)
    _, vals, rows = _sc_topk_gather(scores, cmax, values)
    return _final(rows, vals, param_feats, Wv, w1, b1, w2, b2, gamma, beta)


# speedup vs baseline: 22.1024x; 1.2260x over previous
import functools

import jax
import jax.numpy as jnp
from jax import lax
from jax.experimental import pallas as pl
from jax.experimental.pallas import tpu as pltpu
from jax.experimental.pallas import tpu_sc as plsc

B, N, D, D1, HID = 64, 32768, 128, 32, 64
K_TOP = 16
ROWS_BLK = 4096
NBLK = N // ROWS_BLK
NC, NS, L = 2, 16, 16
NW = NC * NS
QPW = B // NW
CAP = 2048
_HI = lax.Precision.HIGHEST
_DN_NT = (((1,), (1,)), ((), ()))


def _scores_body(q_ref, wq_ref, wk_ref, keys_ref, out_ref, cmax_ref, acc_ref):
    i = pl.program_id(0)
    q1 = lax.dot_general(q_ref[...], wq_ref[...], _DN_NT,
                         preferred_element_type=jnp.float32)
    kc = lax.dot_general(keys_ref[...], wk_ref[...], _DN_NT,
                         preferred_element_type=jnp.float32)
    s = lax.dot_general(q1, kc, _DN_NT, preferred_element_type=jnp.float32)
    out_ref[...] = s
    m = s[:, :D]
    for k in range(1, ROWS_BLK // D):
        m = jnp.maximum(m, s[:, k * D:(k + 1) * D])

    @pl.when(i == 0)
    def _():
        acc_ref[...] = m

    @pl.when(i > 0)
    def _():
        acc_ref[...] = jnp.maximum(acc_ref[...], m)

    @pl.when(i == NBLK - 1)
    def _():
        cmax_ref[...] = acc_ref[...]


def _scores(query, Wq, Wk, keys):
    return pl.pallas_call(
        _scores_body,
        grid=(NBLK,),
        in_specs=[
            pl.BlockSpec((B, D), lambda i: (0, 0)),
            pl.BlockSpec((D, D), lambda i: (0, 0)),
            pl.BlockSpec((D, D), lambda i: (0, 0)),
            pl.BlockSpec((ROWS_BLK, D), lambda i: (i, 0)),
        ],
        out_specs=(pl.BlockSpec((B, ROWS_BLK), lambda i: (0, i)),
                   pl.BlockSpec((B, D), lambda i: (0, 0))),
        out_shape=(jax.ShapeDtypeStruct((B, N), jnp.float32),
                   jax.ShapeDtypeStruct((B, D), jnp.float32)),
        scratch_shapes=[pltpu.VMEM((B, D), jnp.float32)],
    )(query, Wq, Wk, keys)


def _merge16(cv, ci, v, idx):
    sv, si = plsc.sort_key_val(v, idx, descending=False)
    svr = lax.rev(sv, (0,))
    sir = lax.rev(si, (0,))
    nv = jnp.maximum(svr, cv)
    ni = jnp.where(svr >= cv, sir, ci)
    return plsc.sort_key_val(nv, ni, descending=False)


def _sc_body(scores_hbm, cmax_hbm, values_hbm, idx_out, val_out, rows_out,
             srow0, srow1, cmv, cand_v, cand_i, idx_v, val_v, rows_v,
             sem0, sem1, semg):
    wid = lax.axis_index("s") * NC + lax.axis_index("c")
    q0 = wid * QPW
    srows = (srow0, srow1)
    sems = (sem0, sem1)
    copies = [pltpu.make_async_copy(scores_hbm.at[q0 + j], srows[j], sems[j])
              for j in range(QPW)]
    copies[0].start()
    iota = lax.iota(jnp.int32, L)
    ninf = jnp.full((L,), -jnp.inf, jnp.float32)
    zero_i = jnp.zeros((L,), jnp.int32)

    for j in range(QPW):
        q = q0 + j
        srow = srows[j]
        if j + 1 < QPW:
            copies[j + 1].start()

        pltpu.sync_copy(cmax_hbm.at[q], cmv)
        cv, ci = ninf, zero_i
        for k in range(D // L):
            cv, ci = _merge16(cv, ci, cmv[pl.ds(k * L, L)], zero_i)
        tau_vec = -plsc.cummax(lax.rev(plsc.cummax(-cv), (0,)))

        copies[j].wait()

        def scan_step(i, c_vec):
            v = srow[pl.ds(i * L, L)]
            m = v >= tau_vec
            hasany = plsc.all_reduce_population_count(m) > 0
            didx = jnp.minimum(c_vec, CAP - L) + iota
            plsc.store_scatter(cand_v, [didx], v, mask=hasany)
            plsc.store_scatter(cand_i, [didx], iota + i * L, mask=hasany)
            return c_vec + jnp.where(hasany, L, 0)

        c_vec = lax.fori_loop(0, N // L, scan_step, zero_i, unroll=8)
        cnt = jnp.max(c_vec)

        def merge_step(k, carry):
            cv, ci = carry
            v = cand_v[pl.ds(k * L, L)]
            ix = cand_i[pl.ds(k * L, L)]
            return tuple(_merge16(cv, ci, v, ix))

        cur_v, cur_i = lax.fori_loop(0, cnt // L, merge_step,
                                     (ninf, zero_i))

        idx_v[...] = cur_i
        val_v[...] = cur_v
        pltpu.sync_copy(idx_v, idx_out.at[q])
        pltpu.sync_copy(val_v, val_out.at[q])
        pltpu.async_copy(values_hbm.at[idx_v], rows_v, semg).wait()
        pltpu.sync_copy(rows_v, rows_out.at[q])


def _sc_topk_gather(scores, cmax, values):
    mesh = plsc.VectorSubcoreMesh(core_axis_name="c", subcore_axis_name="s",
                                  num_cores=NC, num_subcores=NS)
    fn = pl.kernel(
        _sc_body,
        out_type=(jax.ShapeDtypeStruct((B, K_TOP), jnp.int32),
                  jax.ShapeDtypeStruct((B, K_TOP), jnp.float32),
                  jax.ShapeDtypeStruct((B, K_TOP, D), jnp.float32)),
        mesh=mesh,
        compiler_params=pltpu.CompilerParams(needs_layout_passes=False),
        scratch_types=[
            pltpu.VMEM((N,), jnp.float32),
            pltpu.VMEM((N,), jnp.float32),
            pltpu.VMEM((D,), jnp.float32),
            pltpu.VMEM((CAP,), jnp.float32),
            pltpu.VMEM((CAP,), jnp.int32),
            pltpu.VMEM((K_TOP,), jnp.int32),
            pltpu.VMEM((K_TOP,), jnp.float32),
            pltpu.VMEM((K_TOP, D), jnp.float32),
            pltpu.SemaphoreType.DMA,
            pltpu.SemaphoreType.DMA,
            pltpu.SemaphoreType.DMA,
        ],
    )
    return fn(scores, cmax, values)


def _final_body(rows_ref, sc_ref, pf_ref, wv_ref, w1_ref, b1_ref, w2_ref,
                b2_ref, gamma_ref, beta_ref, out_ref):
    vt = rows_ref[...].reshape(B * K_TOP, D)
    vtop = lax.dot_general(vt, wv_ref[...], _DN_NT,
                           preferred_element_type=jnp.float32, precision=_HI)
    w1 = w1_ref[...]
    h1 = lax.dot_general(vtop, w1[:, :D], _DN_NT,
                         preferred_element_type=jnp.float32, precision=_HI)
    pfh = lax.dot_general(pf_ref[...], w1[:, D:], _DN_NT,
                          preferred_element_type=jnp.float32, precision=_HI)
    pfh = jnp.broadcast_to(pfh[:, None, :], (B, K_TOP, HID)).reshape(
        B * K_TOP, HID)
    h = jnp.maximum(h1 + pfh + b1_ref[...], 0.0)
    h2 = lax.dot_general(h, w2_ref[...], _DN_NT,
                         preferred_element_type=jnp.float32,
                         precision=_HI) + b2_ref[...]
    mu = jnp.mean(h2, axis=1, keepdims=True)
    var = jnp.mean((h2 - mu) * (h2 - mu), axis=1, keepdims=True)
    hn = (h2 - mu) * lax.rsqrt(var + 1e-5) * gamma_ref[...] + beta_ref[...]
    adapted = (vtop + hn).reshape(B, K_TOP, D)

    s = sc_ref[...] * (1.0 / (D ** 0.5))
    e = jnp.exp(s - jnp.max(s, axis=1, keepdims=True))
    w = e / jnp.sum(e, axis=1, keepdims=True)
    out_ref[...] = jnp.sum(adapted * w[:, :, None], axis=1)


def _final(rows, scs, pf, Wv, w1, b1, w2, b2, gamma, beta):
    return pl.pallas_call(
        _final_body,
        out_shape=jax.ShapeDtypeStruct((B, D), jnp.float32),
    )(rows, scs, pf, Wv, w1, b1.reshape(1, HID), w2, b2.reshape(1, D),
      gamma.reshape(1, D), beta.reshape(1, D))


def kernel(query, keys, values, top_k, chunk_size, param_feats,
           Wq, Wk, Wv, w1, b1, w2, b2, gamma, beta):
    if query.ndim == 1:
        query = query[None, :]
    scores, cmax = _scores(query, Wq, Wk, keys)
    _, vals, rows = _sc_topk_gather(scores, cmax, values)
    return _final(rows, vals, param_feats, Wv, w1, b1, w2, b2, gamma, beta)


# speedup vs baseline: 23.8360x; 1.0784x over previous
import functools

import jax
import jax.numpy as jnp
from jax import lax
from jax.experimental import pallas as pl
from jax.experimental.pallas import tpu as pltpu
from jax.experimental.pallas import tpu_sc as plsc

B, N, D, D1, HID = 64, 32768, 128, 32, 64
K_TOP = 16
CHUNK = 128
NCH = N // CHUNK
ROWS_BLK = 4096
NBLK = N // ROWS_BLK
CPB = ROWS_BLK // CHUNK
NC, NS, L = 2, 16, 16
NW = NC * NS
QPW = B // NW
_HI = lax.Precision.HIGHEST
_DN_NT = (((1,), (1,)), ((), ()))


def _scores_body(q_ref, wq_ref, wk_ref, keys_ref, out_ref, cmax_ref):
    q1 = lax.dot_general(q_ref[...], wq_ref[...], _DN_NT,
                         preferred_element_type=jnp.float32)
    kc = lax.dot_general(keys_ref[...], wk_ref[...], _DN_NT,
                         preferred_element_type=jnp.float32)
    s = lax.dot_general(q1, kc, _DN_NT, preferred_element_type=jnp.float32)
    out_ref[...] = s
    cmax_ref[...] = jnp.max(s.reshape(B, CPB, CHUNK), axis=2).reshape(
        1, B, CPB)


def _scores(query, Wq, Wk, keys):
    return pl.pallas_call(
        _scores_body,
        grid=(NBLK,),
        in_specs=[
            pl.BlockSpec((B, D), lambda i: (0, 0)),
            pl.BlockSpec((D, D), lambda i: (0, 0)),
            pl.BlockSpec((D, D), lambda i: (0, 0)),
            pl.BlockSpec((ROWS_BLK, D), lambda i: (i, 0)),
        ],
        out_specs=(pl.BlockSpec((B, ROWS_BLK), lambda i: (0, i)),
                   pl.BlockSpec((1, B, CPB), lambda i: (i, 0, 0))),
        out_shape=(jax.ShapeDtypeStruct((B, N), jnp.float32),
                   jax.ShapeDtypeStruct((NBLK, B, CPB), jnp.float32)),
    )(query, Wq, Wk, keys)


def _merge16(cv, ci, v, idx):
    sv, si = plsc.sort_key_val(v, idx, descending=False)
    svr = lax.rev(sv, (0,))
    sir = lax.rev(si, (0,))
    nv = jnp.maximum(svr, cv)
    ni = jnp.where(svr >= cv, sir, ci)
    return plsc.sort_key_val(nv, ni, descending=False)


def _sc_body(cmax_hbm, scores_hbm, values_hbm, idx_out, val_out, rows_out,
             cmidx, cmbuf, chid, gidx, cbuf, idx_v, val_v, rows_v,
             sem0, sem1):
    wid = lax.axis_index("s") * NC + lax.axis_index("c")
    iota = lax.iota(jnp.int32, L)
    ninf = jnp.full((L,), -jnp.inf, jnp.float32)
    zero_i = jnp.zeros((L,), jnp.int32)

    for j in range(QPW):
        q = wid * QPW + j
        cmidx[...] = jnp.where(iota < NBLK,
                               iota * (B * CPB // CHUNK) + q // 4, 0)
        pltpu.async_copy(cmax_hbm.at[cmidx], cmbuf, sem0).wait()

        cv, ci = ninf, zero_i
        for k in range(NCH // L):
            v = plsc.load_gather(
                cmbuf, [jnp.full((L,), k // 2, jnp.int32),
                        (q % 4) * CPB + (k % 2) * L + iota])
            cv, ci = _merge16(cv, ci, v, k * L + iota)
        tau_vec = -plsc.cummax(lax.rev(plsc.cummax(-cv), (0,)))

        chid[...] = ci
        gidx[...] = ci + q * NCH
        pltpu.async_copy(scores_hbm.at[gidx], cbuf, sem1).wait()

        def scan_step(t, carry):
            g = t // (CHUNK // L)
            r = t % (CHUNK // L)
            v = plsc.load_gather(
                cbuf, [jnp.full((L,), g, jnp.int32), r * L + iota])
            hit = jnp.any(v >= tau_vec)

            def merge(c):
                cid = plsc.load_gather(chid, [jnp.full((L,), g, jnp.int32)])
                nv, ni = _merge16(c[0], c[1], v, cid * CHUNK + r * L + iota)
                return (nv, ni)

            return lax.cond(hit, merge, lambda c: c, carry)

        cur_v, cur_i = lax.fori_loop(
            0, K_TOP * (CHUNK // L), scan_step, (ninf, zero_i), unroll=4)

        idx_v[...] = cur_i
        val_v[...] = cur_v
        pltpu.sync_copy(idx_v, idx_out.at[q])
        pltpu.sync_copy(val_v, val_out.at[q])
        pltpu.async_copy(values_hbm.at[idx_v], rows_v, sem0).wait()
        pltpu.sync_copy(rows_v, rows_out.at[q])


def _sc_topk_gather(cmax, scores, values):
    mesh = plsc.VectorSubcoreMesh(core_axis_name="c", subcore_axis_name="s",
                                  num_cores=NC, num_subcores=NS)
    fn = pl.kernel(
        _sc_body,
        out_type=(jax.ShapeDtypeStruct((B, K_TOP), jnp.int32),
                  jax.ShapeDtypeStruct((B, K_TOP), jnp.float32),
                  jax.ShapeDtypeStruct((B, K_TOP, D), jnp.float32)),
        mesh=mesh,
        compiler_params=pltpu.CompilerParams(needs_layout_passes=False),
        scratch_types=[
            pltpu.VMEM((L,), jnp.int32),
            pltpu.VMEM((L, CHUNK), jnp.float32),
            pltpu.VMEM((K_TOP,), jnp.int32),
            pltpu.VMEM((K_TOP,), jnp.int32),
            pltpu.VMEM((K_TOP, CHUNK), jnp.float32),
            pltpu.VMEM((K_TOP,), jnp.int32),
            pltpu.VMEM((K_TOP,), jnp.float32),
            pltpu.VMEM((K_TOP, D), jnp.float32),
            pltpu.SemaphoreType.DMA,
            pltpu.SemaphoreType.DMA,
        ],
    )
    return fn(cmax, scores, values)


def _final_body(rows_ref, sc_ref, pf_ref, wv_ref, w1_ref, b1_ref, w2_ref,
                b2_ref, gamma_ref, beta_ref, out_ref):
    vt = rows_ref[...].reshape(B * K_TOP, D)
    vtop = lax.dot_general(vt, wv_ref[...], _DN_NT,
                           preferred_element_type=jnp.float32, precision=_HI)
    w1 = w1_ref[...]
    h1 = lax.dot_general(vtop, w1[:, :D], _DN_NT,
                         preferred_element_type=jnp.float32, precision=_HI)
    pfh = lax.dot_general(pf_ref[...], w1[:, D:], _DN_NT,
                          preferred_element_type=jnp.float32, precision=_HI)
    pfh = jnp.broadcast_to(pfh[:, None, :], (B, K_TOP, HID)).reshape(
        B * K_TOP, HID)
    h = jnp.maximum(h1 + pfh + b1_ref[...], 0.0)
    h2 = lax.dot_general(h, w2_ref[...], _DN_NT,
                         preferred_element_type=jnp.float32,
                         precision=_HI) + b2_ref[...]
    mu = jnp.mean(h2, axis=1, keepdims=True)
    var = jnp.mean((h2 - mu) * (h2 - mu), axis=1, keepdims=True)
    hn = (h2 - mu) * lax.rsqrt(var + 1e-5) * gamma_ref[...] + beta_ref[...]
    adapted = (vtop + hn).reshape(B, K_TOP, D)

    s = sc_ref[...] * (1.0 / (D ** 0.5))
    e = jnp.exp(s - jnp.max(s, axis=1, keepdims=True))
    w = e / jnp.sum(e, axis=1, keepdims=True)
    out_ref[...] = jnp.sum(adapted * w[:, :, None], axis=1)


def _final(rows, scs, pf, Wv, w1, b1, w2, b2, gamma, beta):
    return pl.pallas_call(
        _final_body,
        out_shape=jax.ShapeDtypeStruct((B, D), jnp.float32),
    )(rows, scs, pf, Wv, w1, b1.reshape(1, HID), w2, b2.reshape(1, D),
      gamma.reshape(1, D), beta.reshape(1, D))


def kernel(query, keys, values, top_k, chunk_size, param_feats,
           Wq, Wk, Wv, w1, b1, w2, b2, gamma, beta):
    if query.ndim == 1:
        query = query[None, :]
    scores, cmax = _scores(query, Wq, Wk, keys)
    _, vals, rows = _sc_topk_gather(cmax.reshape(NBLK * B * CPB // CHUNK,
                                                 CHUNK),
                                    scores.reshape(B * NCH, CHUNK), values)
    return _final(rows, vals, param_feats, Wv, w1, b1, w2, b2, gamma, beta)


# speedup vs baseline: 39.0521x; 1.6384x over previous
import functools

import jax
import jax.numpy as jnp
from jax import lax
from jax.experimental import pallas as pl
from jax.experimental.pallas import tpu as pltpu
from jax.experimental.pallas import tpu_sc as plsc

B, N, D, D1, HID = 64, 32768, 128, 32, 64
K_TOP = 16
CHUNK = 128
NCH = N // CHUNK
ROWS_BLK = 4096
NBLK = N // ROWS_BLK
CPB = ROWS_BLK // CHUNK
NC, NS, L = 2, 16, 16
NW = NC * NS
QPW = B // NW
_HI = lax.Precision.HIGHEST
_DN_NT = (((1,), (1,)), ((), ()))


def _scores_body(q_ref, wq_ref, wk_ref, keys_ref, out_ref, cmax_ref):
    q1 = lax.dot_general(q_ref[...], wq_ref[...], _DN_NT,
                         preferred_element_type=jnp.float32)
    kc = lax.dot_general(keys_ref[...], wk_ref[...], _DN_NT,
                         preferred_element_type=jnp.float32)
    s = lax.dot_general(q1, kc, _DN_NT, preferred_element_type=jnp.float32)
    s3 = s.reshape(B, CPB, CHUNK)
    out_ref[...] = s3
    cmax_ref[...] = jnp.max(s3, axis=2).reshape(1, B, CPB)


def _scores(query, Wq, Wk, keys):
    return pl.pallas_call(
        _scores_body,
        grid=(NBLK,),
        in_specs=[
            pl.BlockSpec((B, D), lambda i: (0, 0)),
            pl.BlockSpec((D, D), lambda i: (0, 0)),
            pl.BlockSpec((D, D), lambda i: (0, 0)),
            pl.BlockSpec((ROWS_BLK, D), lambda i: (i, 0)),
        ],
        out_specs=(pl.BlockSpec((B, CPB, CHUNK), lambda i: (0, i, 0)),
                   pl.BlockSpec((1, B, CPB), lambda i: (i, 0, 0))),
        out_shape=(jax.ShapeDtypeStruct((B, NCH, CHUNK), jnp.float32),
                   jax.ShapeDtypeStruct((NBLK, B, CPB), jnp.float32)),
    )(query, Wq, Wk, keys)


def _merge16(cv, ci, v, idx):
    sv, si = plsc.sort_key_val(v, idx, descending=False)
    svr = lax.rev(sv, (0,))
    sir = lax.rev(si, (0,))
    nv = jnp.maximum(svr, cv)
    ni = jnp.where(svr >= cv, sir, ci)
    return plsc.sort_key_val(nv, ni, descending=False)


def _sc_body(cmax_hbm, scores_hbm, values_hbm, idx_out, val_out, rows_out,
             cmbuf0, cmbuf1, chid0, chid1, gidx0, gidx1, cbuf0, cbuf1,
             idx_v0, idx_v1, val_v0, val_v1, rows_v0, rows_v1,
             semcm, semc0, semc1, semv0, semv1, semo):
    wid = lax.axis_index("s") * NC + lax.axis_index("c")
    iota = lax.iota(jnp.int32, L)
    ninf = jnp.full((L,), -jnp.inf, jnp.float32)
    zero_i = jnp.zeros((L,), jnp.int32)
    q0 = wid * QPW
    cmbufs = (cmbuf0, cmbuf1)
    chids = (chid0, chid1)
    gidxs = (gidx0, gidx1)
    cbufs = (cbuf0, cbuf1)
    idx_vs = (idx_v0, idx_v1)
    val_vs = (val_v0, val_v1)
    rows_vs = (rows_v0, rows_v1)
    semcs = (semc0, semc1)
    semvs = (semv0, semv1)
    scores2d = scores_hbm

    cm_copies = []
    for j in range(QPW):
        for blk in range(NBLK):
            cp = pltpu.make_async_copy(
                cmax_hbm.at[blk, q0 + j],
                cmbufs[j].at[pl.ds(blk * CPB, CPB)], semcm)
            cp.start()
            cm_copies.append(cp)
    for cp in cm_copies:
        cp.wait()

    taus = []
    for j in range(QPW):
        q = q0 + j
        cv, ci = ninf, zero_i
        for k in range(NCH // L):
            cv, ci = _merge16(cv, ci, cmbufs[j][pl.ds(k * L, L)],
                              k * L + iota)
        taus.append(-plsc.cummax(lax.rev(plsc.cummax(-cv), (0,))))
        chids[j][...] = ci
        gidxs[j][...] = ci + q * NCH
        pltpu.async_copy(scores2d.at[gidxs[j]], cbufs[j], semcs[j]).start()

    out_copies = []
    for j in range(QPW):
        q = q0 + j
        tau_vec = taus[j]
        chid = chids[j]
        cbuf = cbufs[j]
        pltpu.make_async_copy(scores2d.at[gidxs[j]], cbuf, semcs[j]).wait()

        def scan_step(t, carry):
            g = t // (CHUNK // L)
            r = t % (CHUNK // L)
            v = plsc.load_gather(
                cbuf, [jnp.full((L,), g, jnp.int32), r * L + iota])
            hit = jnp.any(v >= tau_vec)

            def merge(c):
                cid = plsc.load_gather(chid, [jnp.full((L,), g, jnp.int32)])
                nv, ni = _merge16(c[0], c[1], v, cid * CHUNK + r * L + iota)
                return (nv, ni)

            return lax.cond(hit, merge, lambda c: c, carry)

        cur_v, cur_i = lax.fori_loop(
            0, K_TOP * (CHUNK // L), scan_step, (ninf, zero_i), unroll=4)

        idx_vs[j][...] = cur_i
        val_vs[j][...] = cur_v
        pltpu.async_copy(values_hbm.at[idx_vs[j]], rows_vs[j],
                         semvs[j]).start()
        for src, dst in ((idx_vs[j], idx_out.at[q]),
                         (val_vs[j], val_out.at[q])):
            cp = pltpu.make_async_copy(src, dst, semo)
            cp.start()
            out_copies.append(cp)

    for j in range(QPW):
        q = q0 + j
        pltpu.make_async_copy(values_hbm.at[idx_vs[j]], rows_vs[j],
                              semvs[j]).wait()
        cp = pltpu.make_async_copy(rows_vs[j], rows_out.at[q], semo)
        cp.start()
        out_copies.append(cp)
    for cp in out_copies:
        cp.wait()


def _sc_topk_gather(cmax, scores, values):
    mesh = plsc.VectorSubcoreMesh(core_axis_name="c", subcore_axis_name="s",
                                  num_cores=NC, num_subcores=NS)
    fn = pl.kernel(
        _sc_body,
        out_type=(jax.ShapeDtypeStruct((B, K_TOP), jnp.int32),
                  jax.ShapeDtypeStruct((B, K_TOP), jnp.float32),
                  jax.ShapeDtypeStruct((B, K_TOP, D), jnp.float32)),
        mesh=mesh,
        compiler_params=pltpu.CompilerParams(needs_layout_passes=False,
                                             use_tc_tiling_on_sc=False),
        scratch_types=(
            [pltpu.VMEM((NCH,), jnp.float32)] * 2
            + [pltpu.VMEM((K_TOP,), jnp.int32)] * 4
            + [pltpu.VMEM((K_TOP, CHUNK), jnp.float32)] * 2
            + [pltpu.VMEM((K_TOP,), jnp.int32)] * 2
            + [pltpu.VMEM((K_TOP,), jnp.float32)] * 2
            + [pltpu.VMEM((K_TOP, D), jnp.float32)] * 2
            + [pltpu.SemaphoreType.DMA] * 6
        ),
    )
    return fn(cmax, scores, values)


def _final_body(rows_ref, sc_ref, pf_ref, wv_ref, w1_ref, b1_ref, w2_ref,
                b2_ref, gamma_ref, beta_ref, out_ref):
    vt = rows_ref[...].reshape(B * K_TOP, D)
    vtop = lax.dot_general(vt, wv_ref[...], _DN_NT,
                           preferred_element_type=jnp.float32, precision=_HI)
    w1 = w1_ref[...]
    h1 = lax.dot_general(vtop, w1[:, :D], _DN_NT,
                         preferred_element_type=jnp.float32, precision=_HI)
    pfh = lax.dot_general(pf_ref[...], w1[:, D:], _DN_NT,
                          preferred_element_type=jnp.float32, precision=_HI)
    pfh = jnp.broadcast_to(pfh[:, None, :], (B, K_TOP, HID)).reshape(
        B * K_TOP, HID)
    h = jnp.maximum(h1 + pfh + b1_ref[...], 0.0)
    h2 = lax.dot_general(h, w2_ref[...], _DN_NT,
                         preferred_element_type=jnp.float32,
                         precision=_HI) + b2_ref[...]
    mu = jnp.mean(h2, axis=1, keepdims=True)
    var = jnp.mean((h2 - mu) * (h2 - mu), axis=1, keepdims=True)
    hn = (h2 - mu) * lax.rsqrt(var + 1e-5) * gamma_ref[...] + beta_ref[...]
    adapted = (vtop + hn).reshape(B, K_TOP, D)

    s = sc_ref[...] * (1.0 / (D ** 0.5))
    e = jnp.exp(s - jnp.max(s, axis=1, keepdims=True))
    w = e / jnp.sum(e, axis=1, keepdims=True)
    out_ref[...] = jnp.sum(adapted * w[:, :, None], axis=1)


def _final(rows, scs, pf, Wv, w1, b1, w2, b2, gamma, beta):
    return pl.pallas_call(
        _final_body,
        out_shape=jax.ShapeDtypeStruct((B, D), jnp.float32),
    )(rows, scs, pf, Wv, w1, b1.reshape(1, HID), w2, b2.reshape(1, D),
      gamma.reshape(1, D), beta.reshape(1, D))


def kernel(query, keys, values, top_k, chunk_size, param_feats,
           Wq, Wk, Wv, w1, b1, w2, b2, gamma, beta):
    if query.ndim == 1:
        query = query[None, :]
    scores, cmax = _scores(query, Wq, Wk, keys)
    _, vals, rows = _sc_topk_gather(cmax, scores.reshape(B * NCH, CHUNK),
                                    values)
    return _final(rows, vals, param_feats, Wv, w1, b1, w2, b2, gamma, beta)


# speedup vs baseline: 41.6166x; 1.0657x over previous
import functools

import jax
import jax.numpy as jnp
from jax import lax
from jax.experimental import pallas as pl
from jax.experimental.pallas import tpu as pltpu
from jax.experimental.pallas import tpu_sc as plsc

B, N, D, D1, HID = 64, 32768, 128, 32, 64
K_TOP = 16
CHUNK = 128
NCH = N // CHUNK
ROWS_BLK = 8192
NBLK = N // ROWS_BLK
CPB = ROWS_BLK // CHUNK
NC, NS, L = 2, 16, 16
NW = NC * NS
QPW = B // NW
_HI = lax.Precision.HIGHEST
_DN_NT = (((1,), (1,)), ((), ()))


def _scores_body(q_ref, wq_ref, wk_ref, keys_ref, out_ref, cmax_ref):
    q1 = lax.dot_general(q_ref[...], wq_ref[...], _DN_NT,
                         preferred_element_type=jnp.float32)
    kc = lax.dot_general(keys_ref[...], wk_ref[...], _DN_NT,
                         preferred_element_type=jnp.float32)
    s = lax.dot_general(q1, kc, _DN_NT, preferred_element_type=jnp.float32)
    s3 = s.reshape(B, CPB, CHUNK)
    out_ref[...] = s3
    cmax_ref[...] = jnp.max(s3, axis=2).reshape(1, B, CPB)


def _scores(query, Wq, Wk, keys):
    return pl.pallas_call(
        _scores_body,
        grid=(NBLK,),
        in_specs=[
            pl.BlockSpec((B, D), lambda i: (0, 0)),
            pl.BlockSpec((D, D), lambda i: (0, 0)),
            pl.BlockSpec((D, D), lambda i: (0, 0)),
            pl.BlockSpec((ROWS_BLK, D), lambda i: (i, 0)),
        ],
        out_specs=(pl.BlockSpec((B, CPB, CHUNK), lambda i: (0, i, 0)),
                   pl.BlockSpec((1, B, CPB), lambda i: (i, 0, 0))),
        out_shape=(jax.ShapeDtypeStruct((B, NCH, CHUNK), jnp.float32),
                   jax.ShapeDtypeStruct((NBLK, B, CPB), jnp.float32)),
    )(query, Wq, Wk, keys)


def _merge16(cv, ci, v, idx):
    sv, si = plsc.sort_key_val(v, idx, descending=False)
    svr = lax.rev(sv, (0,))
    sir = lax.rev(si, (0,))
    nv = jnp.maximum(svr, cv)
    ni = jnp.where(svr >= cv, sir, ci)
    return plsc.sort_key_val(nv, ni, descending=False)


def _sc_body(cmax_hbm, scores_hbm, values_hbm, idx_out, val_out, rows_out,
             cmbuf0, cmbuf1, chid0, chid1, gidx0, gidx1, cbuf0, cbuf1,
             idx_v0, idx_v1, val_v0, val_v1, rows_v0, rows_v1,
             semcm, semc0, semc1, semv0, semv1, semo):
    wid = lax.axis_index("s") * NC + lax.axis_index("c")
    iota = lax.iota(jnp.int32, L)
    ninf = jnp.full((L,), -jnp.inf, jnp.float32)
    zero_i = jnp.zeros((L,), jnp.int32)
    q0 = wid * QPW
    cmbufs = (cmbuf0, cmbuf1)
    chids = (chid0, chid1)
    gidxs = (gidx0, gidx1)
    cbufs = (cbuf0, cbuf1)
    idx_vs = (idx_v0, idx_v1)
    val_vs = (val_v0, val_v1)
    rows_vs = (rows_v0, rows_v1)
    semcs = (semc0, semc1)
    semvs = (semv0, semv1)
    scores2d = scores_hbm

    cm_copies = []
    for j in range(QPW):
        for blk in range(NBLK):
            cp = pltpu.make_async_copy(
                cmax_hbm.at[blk, q0 + j],
                cmbufs[j].at[pl.ds(blk * CPB, CPB)], semcm)
            cp.start()
            cm_copies.append(cp)
    for cp in cm_copies:
        cp.wait()

    taus = []
    for j in range(QPW):
        q = q0 + j
        cv, ci = ninf, zero_i
        for k in range(NCH // L):
            cv, ci = _merge16(cv, ci, cmbufs[j][pl.ds(k * L, L)],
                              k * L + iota)
        taus.append(-plsc.cummax(lax.rev(plsc.cummax(-cv), (0,))))
        chids[j][...] = ci
        gidxs[j][...] = ci + q * NCH
        pltpu.async_copy(scores2d.at[gidxs[j]], cbufs[j], semcs[j]).start()

    out_copies = []
    for j in range(QPW):
        q = q0 + j
        tau_vec = taus[j]
        chid = chids[j]
        cbuf = cbufs[j]
        pltpu.make_async_copy(scores2d.at[gidxs[j]], cbuf, semcs[j]).wait()

        def scan_step(t, carry):
            g = t // (CHUNK // L)
            r = t % (CHUNK // L)
            v = plsc.load_gather(
                cbuf, [jnp.full((L,), g, jnp.int32), r * L + iota])
            hit = jnp.any(v >= tau_vec)

            def merge(c):
                cid = plsc.load_gather(chid, [jnp.full((L,), g, jnp.int32)])
                nv, ni = _merge16(c[0], c[1], v, cid * CHUNK + r * L + iota)
                return (nv, ni)

            return lax.cond(hit, merge, lambda c: c, carry)

        cur_v, cur_i = lax.fori_loop(
            0, K_TOP * (CHUNK // L), scan_step, (ninf, zero_i), unroll=4)

        idx_vs[j][...] = cur_i
        val_vs[j][...] = cur_v
        pltpu.async_copy(values_hbm.at[idx_vs[j]], rows_vs[j],
                         semvs[j]).start()
        for src, dst in ((idx_vs[j], idx_out.at[q]),
                         (val_vs[j], val_out.at[q])):
            cp = pltpu.make_async_copy(src, dst, semo)
            cp.start()
            out_copies.append(cp)

    for j in range(QPW):
        q = q0 + j
        pltpu.make_async_copy(values_hbm.at[idx_vs[j]], rows_vs[j],
                              semvs[j]).wait()
        cp = pltpu.make_async_copy(rows_vs[j], rows_out.at[q], semo)
        cp.start()
        out_copies.append(cp)
    for cp in out_copies:
        cp.wait()


def _sc_topk_gather(cmax, scores, values):
    mesh = plsc.VectorSubcoreMesh(core_axis_name="c", subcore_axis_name="s",
                                  num_cores=NC, num_subcores=NS)
    fn = pl.kernel(
        _sc_body,
        out_type=(jax.ShapeDtypeStruct((B, K_TOP), jnp.int32),
                  jax.ShapeDtypeStruct((B, K_TOP), jnp.float32),
                  jax.ShapeDtypeStruct((B, K_TOP, D), jnp.float32)),
        mesh=mesh,
        compiler_params=pltpu.CompilerParams(needs_layout_passes=False,
                                             use_tc_tiling_on_sc=False),
        scratch_types=(
            [pltpu.VMEM((NCH,), jnp.float32)] * 2
            + [pltpu.VMEM((K_TOP,), jnp.int32)] * 4
            + [pltpu.VMEM((K_TOP, CHUNK), jnp.float32)] * 2
            + [pltpu.VMEM((K_TOP,), jnp.int32)] * 2
            + [pltpu.VMEM((K_TOP,), jnp.float32)] * 2
            + [pltpu.VMEM((K_TOP, D), jnp.float32)] * 2
            + [pltpu.SemaphoreType.DMA] * 6
        ),
    )
    return fn(cmax, scores, values)


def _final_body(rows_ref, sc_ref, pf_ref, wv_ref, w1_ref, b1_ref, w2_ref,
                b2_ref, gamma_ref, beta_ref, out_ref):
    vt = rows_ref[...].reshape(B * K_TOP, D)
    vtop = lax.dot_general(vt, wv_ref[...], _DN_NT,
                           preferred_element_type=jnp.float32, precision=_HI)
    w1 = w1_ref[...]
    h1 = lax.dot_general(vtop, w1[:, :D], _DN_NT,
                         preferred_element_type=jnp.float32, precision=_HI)
    pfh = lax.dot_general(pf_ref[...], w1[:, D:], _DN_NT,
                          preferred_element_type=jnp.float32, precision=_HI)
    pfh = jnp.broadcast_to(pfh[:, None, :], (B, K_TOP, HID)).reshape(
        B * K_TOP, HID)
    h = jnp.maximum(h1 + pfh + b1_ref[...], 0.0)
    h2 = lax.dot_general(h, w2_ref[...], _DN_NT,
                         preferred_element_type=jnp.float32,
                         precision=_HI) + b2_ref[...]
    mu = jnp.mean(h2, axis=1, keepdims=True)
    var = jnp.mean((h2 - mu) * (h2 - mu), axis=1, keepdims=True)
    hn = (h2 - mu) * lax.rsqrt(var + 1e-5) * gamma_ref[...] + beta_ref[...]
    adapted = (vtop + hn).reshape(B, K_TOP, D)

    s = sc_ref[...] * (1.0 / (D ** 0.5))
    e = jnp.exp(s - jnp.max(s, axis=1, keepdims=True))
    w = e / jnp.sum(e, axis=1, keepdims=True)
    out_ref[...] = jnp.sum(adapted * w[:, :, None], axis=1)


def _final(rows, scs, pf, Wv, w1, b1, w2, b2, gamma, beta):
    return pl.pallas_call(
        _final_body,
        out_shape=jax.ShapeDtypeStruct((B, D), jnp.float32),
    )(rows, scs, pf, Wv, w1, b1.reshape(1, HID), w2, b2.reshape(1, D),
      gamma.reshape(1, D), beta.reshape(1, D))


def kernel(query, keys, values, top_k, chunk_size, param_feats,
           Wq, Wk, Wv, w1, b1, w2, b2, gamma, beta):
    if query.ndim == 1:
        query = query[None, :]
    scores, cmax = _scores(query, Wq, Wk, keys)
    _, vals, rows = _sc_topk_gather(cmax, scores.reshape(B * NCH, CHUNK),
                                    values)
    return _final(rows, vals, param_feats, Wv, w1, b1, w2, b2, gamma, beta)


# speedup vs baseline: 43.7478x; 1.0512x over previous
import functools

import jax
import jax.numpy as jnp
from jax import lax
from jax.experimental import pallas as pl
from jax.experimental.pallas import tpu as pltpu
from jax.experimental.pallas import tpu_sc as plsc

B, N, D, D1, HID = 64, 32768, 128, 32, 64
K_TOP = 16
CHUNK = 128
NCH = N // CHUNK
ROWS_BLK = 16384
NBLK = N // ROWS_BLK
CPB = ROWS_BLK // CHUNK
NC, NS, L = 2, 16, 16
NW = NC * NS
QPW = B // NW
_HI = lax.Precision.HIGHEST
_DN_NT = (((1,), (1,)), ((), ()))


def _scores_body(q_ref, wq_ref, wk_ref, keys_ref, out_ref, cmax_ref):
    q1 = lax.dot_general(q_ref[...], wq_ref[...], _DN_NT,
                         preferred_element_type=jnp.float32)
    kc = lax.dot_general(keys_ref[...], wk_ref[...], _DN_NT,
                         preferred_element_type=jnp.float32)
    s = lax.dot_general(q1, kc, _DN_NT, preferred_element_type=jnp.float32)
    s3 = s.reshape(B, CPB, CHUNK)
    out_ref[...] = s3
    cmax_ref[...] = jnp.max(s3, axis=2).reshape(1, B, CPB)


def _scores(query, Wq, Wk, keys):
    return pl.pallas_call(
        _scores_body,
        grid=(NBLK,),
        in_specs=[
            pl.BlockSpec((B, D), lambda i: (0, 0)),
            pl.BlockSpec((D, D), lambda i: (0, 0)),
            pl.BlockSpec((D, D), lambda i: (0, 0)),
            pl.BlockSpec((ROWS_BLK, D), lambda i: (i, 0)),
        ],
        out_specs=(pl.BlockSpec((B, CPB, CHUNK), lambda i: (0, i, 0)),
                   pl.BlockSpec((1, B, CPB), lambda i: (i, 0, 0))),
        out_shape=(jax.ShapeDtypeStruct((B, NCH, CHUNK), jnp.float32),
                   jax.ShapeDtypeStruct((NBLK, B, CPB), jnp.float32)),
    )(query, Wq, Wk, keys)


def _merge16(cv, ci, v, idx):
    sv, si = plsc.sort_key_val(v, idx, descending=False)
    svr = lax.rev(sv, (0,))
    sir = lax.rev(si, (0,))
    nv = jnp.maximum(svr, cv)
    ni = jnp.where(svr >= cv, sir, ci)
    return plsc.sort_key_val(nv, ni, descending=False)


def _sc_body(cmax_hbm, scores_hbm, values_hbm, idx_out, val_out, rows_out,
             cmbuf0, cmbuf1, chid0, chid1, gidx0, gidx1, cbuf0, cbuf1,
             idx_v0, idx_v1, val_v0, val_v1, rows_v0, rows_v1,
             semcm, semc0, semc1, semv0, semv1, semo):
    wid = lax.axis_index("s") * NC + lax.axis_index("c")
    iota = lax.iota(jnp.int32, L)
    ninf = jnp.full((L,), -jnp.inf, jnp.float32)
    zero_i = jnp.zeros((L,), jnp.int32)
    q0 = wid * QPW
    cmbufs = (cmbuf0, cmbuf1)
    chids = (chid0, chid1)
    gidxs = (gidx0, gidx1)
    cbufs = (cbuf0, cbuf1)
    idx_vs = (idx_v0, idx_v1)
    val_vs = (val_v0, val_v1)
    rows_vs = (rows_v0, rows_v1)
    semcs = (semc0, semc1)
    semvs = (semv0, semv1)
    scores2d = scores_hbm

    cm_copies = []
    for j in range(QPW):
        for blk in range(NBLK):
            cp = pltpu.make_async_copy(
                cmax_hbm.at[blk, q0 + j],
                cmbufs[j].at[pl.ds(blk * CPB, CPB)], semcm)
            cp.start()
            cm_copies.append(cp)
    for cp in cm_copies:
        cp.wait()

    taus = []
    for j in range(QPW):
        q = q0 + j
        cv, ci = ninf, zero_i
        for k in range(NCH // L):
            cv, ci = _merge16(cv, ci, cmbufs[j][pl.ds(k * L, L)],
                              k * L + iota)
        taus.append(-plsc.cummax(lax.rev(plsc.cummax(-cv), (0,))))
        chids[j][...] = ci
        gidxs[j][...] = ci + q * NCH
        pltpu.async_copy(scores2d.at[gidxs[j]], cbufs[j], semcs[j]).start()

    out_copies = []
    for j in range(QPW):
        q = q0 + j
        tau_vec = taus[j]
        chid = chids[j]
        cbuf = cbufs[j]
        pltpu.make_async_copy(scores2d.at[gidxs[j]], cbuf, semcs[j]).wait()

        def scan_step(t, carry):
            g = t // (CHUNK // L)
            r = t % (CHUNK // L)
            v = plsc.load_gather(
                cbuf, [jnp.full((L,), g, jnp.int32), r * L + iota])
            hit = jnp.any(v >= tau_vec)

            def merge(c):
                cid = plsc.load_gather(chid, [jnp.full((L,), g, jnp.int32)])
                nv, ni = _merge16(c[0], c[1], v, cid * CHUNK + r * L + iota)
                return (nv, ni)

            return lax.cond(hit, merge, lambda c: c, carry)

        cur_v, cur_i = lax.fori_loop(
            0, K_TOP * (CHUNK // L), scan_step, (ninf, zero_i), unroll=4)

        idx_vs[j][...] = cur_i
        val_vs[j][...] = cur_v
        pltpu.async_copy(values_hbm.at[idx_vs[j]], rows_vs[j],
                         semvs[j]).start()
        for src, dst in ((idx_vs[j], idx_out.at[q]),
                         (val_vs[j], val_out.at[q])):
            cp = pltpu.make_async_copy(src, dst, semo)
            cp.start()
            out_copies.append(cp)

    for j in range(QPW):
        q = q0 + j
        pltpu.make_async_copy(values_hbm.at[idx_vs[j]], rows_vs[j],
                              semvs[j]).wait()
        cp = pltpu.make_async_copy(rows_vs[j], rows_out.at[q], semo)
        cp.start()
        out_copies.append(cp)
    for cp in out_copies:
        cp.wait()


def _sc_topk_gather(cmax, scores, values):
    mesh = plsc.VectorSubcoreMesh(core_axis_name="c", subcore_axis_name="s",
                                  num_cores=NC, num_subcores=NS)
    fn = pl.kernel(
        _sc_body,
        out_type=(jax.ShapeDtypeStruct((B, K_TOP), jnp.int32),
                  jax.ShapeDtypeStruct((B, K_TOP), jnp.float32),
                  jax.ShapeDtypeStruct((B, K_TOP, D), jnp.float32)),
        mesh=mesh,
        compiler_params=pltpu.CompilerParams(needs_layout_passes=False,
                                             use_tc_tiling_on_sc=False),
        scratch_types=(
            [pltpu.VMEM((NCH,), jnp.float32)] * 2
            + [pltpu.VMEM((K_TOP,), jnp.int32)] * 4
            + [pltpu.VMEM((K_TOP, CHUNK), jnp.float32)] * 2
            + [pltpu.VMEM((K_TOP,), jnp.int32)] * 2
            + [pltpu.VMEM((K_TOP,), jnp.float32)] * 2
            + [pltpu.VMEM((K_TOP, D), jnp.float32)] * 2
            + [pltpu.SemaphoreType.DMA] * 6
        ),
    )
    return fn(cmax, scores, values)


def _final_body(rows_ref, sc_ref, pf_ref, wv_ref, w1_ref, b1_ref, w2_ref,
                b2_ref, gamma_ref, beta_ref, out_ref):
    vt = rows_ref[...].reshape(B * K_TOP, D)
    vtop = lax.dot_general(vt, wv_ref[...], _DN_NT,
                           preferred_element_type=jnp.float32, precision=_HI)
    w1 = w1_ref[...]
    h1 = lax.dot_general(vtop, w1[:, :D], _DN_NT,
                         preferred_element_type=jnp.float32, precision=_HI)
    pfh = lax.dot_general(pf_ref[...], w1[:, D:], _DN_NT,
                          preferred_element_type=jnp.float32, precision=_HI)
    pfh = jnp.broadcast_to(pfh[:, None, :], (B, K_TOP, HID)).reshape(
        B * K_TOP, HID)
    h = jnp.maximum(h1 + pfh + b1_ref[...], 0.0)
    h2 = lax.dot_general(h, w2_ref[...], _DN_NT,
                         preferred_element_type=jnp.float32,
                         precision=_HI) + b2_ref[...]
    mu = jnp.mean(h2, axis=1, keepdims=True)
    var = jnp.mean((h2 - mu) * (h2 - mu), axis=1, keepdims=True)
    hn = (h2 - mu) * lax.rsqrt(var + 1e-5) * gamma_ref[...] + beta_ref[...]
    adapted = (vtop + hn).reshape(B, K_TOP, D)

    s = sc_ref[...] * (1.0 / (D ** 0.5))
    e = jnp.exp(s - jnp.max(s, axis=1, keepdims=True))
    w = e / jnp.sum(e, axis=1, keepdims=True)
    out_ref[...] = jnp.sum(adapted * w[:, :, None], axis=1)


def _final(rows, scs, pf, Wv, w1, b1, w2, b2, gamma, beta):
    return pl.pallas_call(
        _final_body,
        out_shape=jax.ShapeDtypeStruct((B, D), jnp.float32),
    )(rows, scs, pf, Wv, w1, b1.reshape(1, HID), w2, b2.reshape(1, D),
      gamma.reshape(1, D), beta.reshape(1, D))


def kernel(query, keys, values, top_k, chunk_size, param_feats,
           Wq, Wk, Wv, w1, b1, w2, b2, gamma, beta):
    if query.ndim == 1:
        query = query[None, :]
    scores, cmax = _scores(query, Wq, Wk, keys)
    _, vals, rows = _sc_topk_gather(cmax, scores.reshape(B * NCH, CHUNK),
                                    values)
    return _final(rows, vals, param_feats, Wv, w1, b1, w2, b2, gamma, beta)
